# baseline (speedup 1.0000x reference)
import functools

import jax
import jax.numpy as jnp
from jax import lax
from jax.experimental import pallas as pl
from jax.experimental.pallas import tpu as pltpu
from jax.experimental.pallas import tpu_sc as plsc

N = 65536
L = 10
V = 10
ROW = L * V
NC = 2
NS = 16
NW = NC * NS
ROWS_PER_W = N // NW
SLAB = ROWS_PER_W * ROW
CHUNK_ROWS = 256
CHUNKS = ROWS_PER_W // CHUNK_ROWS
GROUPS = CHUNK_ROWS // 16
CHUNK_ELEMS = CHUNK_ROWS * ROW


def _ci(v):
    return lax.full((16,), v, jnp.int32)


def _cf(v):
    return lax.full((16,), v, jnp.float32)


@functools.partial(
    pl.kernel,
    out_type=jax.ShapeDtypeStruct((NW, SLAB), jnp.float32),
    mesh=plsc.VectorSubcoreMesh(core_axis_name="c", subcore_axis_name="s"),
    scratch_types=[pltpu.VMEM((CHUNK_ELEMS,), jnp.float32)],
    compiler_params=pltpu.CompilerParams(needs_layout_passes=False),
)
def _solve(x_hbm, out_hbm, buf):
    wid = lax.axis_index("s") * NC + lax.axis_index("c")
    lanes = lax.iota(jnp.int32, 16) * ROW

    def chunk_body(ci, carry):
        off = ci * CHUNK_ELEMS
        pltpu.sync_copy(x_hbm.at[wid, pl.ds(off, CHUNK_ELEMS)], buf)

        def group_body(gi, c2):
            idx0 = lanes + gi * (16 * ROW)
            orall = _ci(0)
            for l in range(L):
                gbase = idx0 + (l * V)
                m = plsc.load_gather(buf, [gbase])
                b = _ci(1)
                for v in range(1, V):
                    xv = plsc.load_gather(buf, [gbase + v])
                    gt = xv > m
                    b = jnp.where(gt, _ci(1 << v), b)
                    m = jnp.where(gt, xv, m)
                orall = orall | b
            one = _cf(1.0)
            zero = _cf(0.0)
            for l in range(1, L):
                miss = (orall & _ci(1 << (l - 1))) == _ci(0)
                for v in range(V):
                    val = one if v == (l - 1) else zero
                    plsc.store_scatter(buf, [idx0 + (l * V + v)], val,
                                       mask=miss)
            return c2

        lax.fori_loop(0, GROUPS, group_body, 0)
        pltpu.sync_copy(buf, out_hbm.at[wid, pl.ds(off, CHUNK_ELEMS)])
        return carry

    lax.fori_loop(0, CHUNKS, chunk_body, 0)


def kernel(memory_vb):
    out = _solve(memory_vb.reshape(NW, SLAB))
    return out.reshape(memory_vb.shape)


# speedup vs baseline: 9.5738x; 9.5738x over previous
import functools

import jax
import jax.numpy as jnp
from jax import lax
from jax.experimental import pallas as pl
from jax.experimental.pallas import tpu as pltpu
from jax.experimental.pallas import tpu_sc as plsc

N = 65536
L = 10
V = 10
P = L * V
NC = 2
NS = 16
NW = NC * NS
N_PER_W = N // NW
CHUNK_N = 1024
CHUNKS = N_PER_W // CHUNK_N
GROUPS = CHUNK_N // 16


def _ci(v):
    return lax.full((16,), v, jnp.int32)


def _cf(v):
    return lax.full((16,), v, jnp.float32)


@functools.partial(
    pl.kernel,
    out_type=jax.ShapeDtypeStruct((P, N), jnp.float32),
    mesh=plsc.VectorSubcoreMesh(core_axis_name="c", subcore_axis_name="s"),
    scratch_types=[pltpu.VMEM((P, CHUNK_N), jnp.float32)],
    compiler_params=pltpu.CompilerParams(needs_layout_passes=False),
)
def _solve(x_hbm, out_hbm, buf):
    wid = lax.axis_index("s") * NC + lax.axis_index("c")
    base_n = wid * N_PER_W
    lanes = lax.iota(jnp.int32, 16)

    def chunk_body(ci, carry):
        n0 = base_n + ci * CHUNK_N
        pltpu.sync_copy(x_hbm.at[:, pl.ds(n0, CHUNK_N)], buf)

        def group_body(gi, c2):
            g0 = gi * 16
            orall = _ci(0)
            for l in range(L):
                m = buf[l * V, pl.ds(g0, 16)]
                b = _ci(1)
                for v in range(1, V):
                    xv = buf[l * V + v, pl.ds(g0, 16)]
                    gt = xv > m
                    b = jnp.where(gt, _ci(1 << v), b)
                    m = jnp.where(gt, xv, m)
                orall = orall | b
            nidx = lanes + g0
            one = _cf(1.0)
            zero = _cf(0.0)
            for l in range(1, L):
                miss = (orall & _ci(1 << (l - 1))) == _ci(0)
                for v in range(V):
                    val = one if v == (l - 1) else zero
                    plsc.store_scatter(buf, [_ci(l * V + v), nidx], val,
                                       mask=miss)
            return c2

        lax.fori_loop(0, GROUPS, group_body, 0)
        pltpu.sync_copy(buf, out_hbm.at[:, pl.ds(n0, CHUNK_N)])
        return carry

    lax.fori_loop(0, CHUNKS, chunk_body, 0)


def kernel(memory_vb):
    xt = memory_vb.transpose(1, 2, 0).reshape(P, N)
    out = _solve(xt)
    return out.reshape(L, V, N).transpose(2, 0, 1)


# speedup vs baseline: 16.1078x; 1.6825x over previous
import functools

import jax
import jax.numpy as jnp
from jax import lax
from jax.experimental import pallas as pl
from jax.experimental.pallas import tpu as pltpu
from jax.experimental.pallas import tpu_sc as plsc

N = 65536
L = 10
V = 10
P = L * V
NC = 2
NS = 16
NW = NC * NS
N_PER_W = N // NW
CHUNK_N = 512
CHUNKS = N_PER_W // CHUNK_N
GROUPS = CHUNK_N // 16


def _ci(v):
    return lax.full((16,), v, jnp.int32)


def _cf(v):
    return lax.full((16,), v, jnp.float32)


@functools.partial(
    pl.kernel,
    out_type=jax.ShapeDtypeStruct((L, V, N), jnp.float32),
    mesh=plsc.VectorSubcoreMesh(core_axis_name="c", subcore_axis_name="s"),
    scratch_types=[pltpu.VMEM((L, V, CHUNK_N), jnp.float32)],
    compiler_params=pltpu.CompilerParams(needs_layout_passes=False,
                                         use_tc_tiling_on_sc=True),
)
def _solve(x_hbm, out_hbm, buf):
    wid = lax.axis_index("s") * NC + lax.axis_index("c")
    base_n = wid * N_PER_W
    lanes = lax.iota(jnp.int32, 16)

    def chunk_body(ci, carry):
        n0 = base_n + ci * CHUNK_N
        pltpu.sync_copy(x_hbm.at[:, :, pl.ds(n0, CHUNK_N)], buf)

        def group_body(gi, c2):
            g0 = gi * 16
            orall = _ci(0)
            for l in range(L):
                m = buf[l, 0, pl.ds(g0, 16)]
                b = _ci(1)
                for v in range(1, V):
                    xv = buf[l, v, pl.ds(g0, 16)]
                    gt = xv > m
                    b = jnp.where(gt, _ci(1 << v), b)
                    m = jnp.where(gt, xv, m)
                orall = orall | b
            nidx = lanes + g0
            one = _cf(1.0)
            zero = _cf(0.0)
            for l in range(1, L):
                miss = (orall & _ci(1 << (l - 1))) == _ci(0)
                for v in range(V):
                    val = one if v == (l - 1) else zero
                    plsc.store_scatter(buf, [_ci(l), _ci(v), nidx], val,
                                       mask=miss)
            return c2

        lax.fori_loop(0, GROUPS, group_body, 0)
        pltpu.sync_copy(buf, out_hbm.at[:, :, pl.ds(n0, CHUNK_N)])
        return carry

    lax.fori_loop(0, CHUNKS, chunk_body, 0)


def kernel(memory_vb):
    xt = memory_vb.transpose(1, 2, 0)
    out = _solve(xt)
    return out.transpose(2, 0, 1)


# speedup vs baseline: 18.7689x; 1.1652x over previous
import functools

import jax
import jax.numpy as jnp
from jax import lax
from jax.experimental import pallas as pl
from jax.experimental.pallas import tpu as pltpu
from jax.experimental.pallas import tpu_sc as plsc

N = 65536
L = 10
V = 10
NC = 2
NS = 16
NW = NC * NS
N_PER_W = N // NW
CHUNK_N = 256
CHUNKS = N_PER_W // CHUNK_N
GROUPS = CHUNK_N // 16


def _ci(v):
    return lax.full((16,), v, jnp.int32)


def _cf(v):
    return lax.full((16,), v, jnp.float32)


@functools.partial(
    pl.kernel,
    out_type=jax.ShapeDtypeStruct((L, V, N), jnp.float32),
    mesh=plsc.VectorSubcoreMesh(core_axis_name="c", subcore_axis_name="s"),
    scratch_types=[
        pltpu.VMEM((L, V, CHUNK_N), jnp.float32),
        pltpu.VMEM((L, V, CHUNK_N), jnp.float32),
        pltpu.SemaphoreType.DMA,
        pltpu.SemaphoreType.DMA,
        pltpu.SemaphoreType.DMA,
        pltpu.SemaphoreType.DMA,
    ],
    compiler_params=pltpu.CompilerParams(needs_layout_passes=False,
                                         use_tc_tiling_on_sc=True),
)
def _solve(x_hbm, out_hbm, buf0, buf1, si0, si1, so0, so1):
    bufs = (buf0, buf1)
    sin = (si0, si1)
    sout = (so0, so1)
    wid = lax.axis_index("s") * NC + lax.axis_index("c")
    base_n = wid * N_PER_W
    lanes = lax.iota(jnp.int32, 16)

    def src(ci):
        return x_hbm.at[:, :, pl.ds(base_n + ci * CHUNK_N, CHUNK_N)]

    def dst(ci):
        return out_hbm.at[:, :, pl.ds(base_n + ci * CHUNK_N, CHUNK_N)]

    def compute(buf):
        def group_body(gi, c2):
            g0 = gi * 16
            orall = _ci(0)
            for l in range(L):
                m = buf[l, 0, pl.ds(g0, 16)]
                b = _ci(1)
                for v in range(1, V):
                    xv = buf[l, v, pl.ds(g0, 16)]
                    gt = xv > m
                    b = jnp.where(gt, _ci(1 << v), b)
                    m = jnp.where(gt, xv, m)
                orall = orall | b
            nidx = lanes + g0
            one = _cf(1.0)
            zero = _cf(0.0)
            for l in range(1, L):
                miss = (orall & _ci(1 << (l - 1))) == _ci(0)
                for v in range(V):
                    val = one if v == (l - 1) else zero
                    plsc.store_scatter(buf, [_ci(l), _ci(v), nidx], val,
                                       mask=miss)
            return c2

        lax.fori_loop(0, GROUPS, group_body, 0)

    pltpu.async_copy(src(0), bufs[0], sin[0])
    for ci in range(CHUNKS):
        b = ci % 2
        nb = (ci + 1) % 2
        if ci + 1 < CHUNKS:
            if ci >= 1:
                pltpu.make_async_copy(bufs[nb], dst(ci - 1), sout[nb]).wait()
            pltpu.async_copy(src(ci + 1), bufs[nb], sin[nb])
        pltpu.make_async_copy(src(ci), bufs[b], sin[b]).wait()
        compute(bufs[b])
        pltpu.async_copy(bufs[b], dst(ci), sout[b])
    pltpu.make_async_copy(bufs[(CHUNKS - 2) % 2], dst(CHUNKS - 2),
                          sout[(CHUNKS - 2) % 2]).wait()
    pltpu.make_async_copy(bufs[(CHUNKS - 1) % 2], dst(CHUNKS - 1),
                          sout[(CHUNKS - 1) % 2]).wait()


def kernel(memory_vb):
    xt = memory_vb.transpose(1, 2, 0)
    out = _solve(xt)
    return out.transpose(2, 0, 1)


# speedup vs baseline: 19.2845x; 1.0275x over previous
import functools

import jax
import jax.numpy as jnp
from jax import lax
from jax.experimental import pallas as pl
from jax.experimental.pallas import tpu as pltpu
from jax.experimental.pallas import tpu_sc as plsc

N = 65536
L = 10
V = 10
NC = 2
NS = 16
NW = NC * NS
N_PER_W = N // NW
CHUNK_N = 256
CHUNKS = N_PER_W // CHUNK_N
GROUPS = CHUNK_N // 16


def _ci(v):
    return lax.full((16,), v, jnp.int32)


def _cf(v):
    return lax.full((16,), v, jnp.float32)


@functools.partial(
    pl.kernel,
    out_type=jax.ShapeDtypeStruct((L, V, N), jnp.float32),
    mesh=plsc.VectorSubcoreMesh(core_axis_name="c", subcore_axis_name="s"),
    scratch_types=[
        pltpu.VMEM((L, V, CHUNK_N), jnp.float32),
        pltpu.VMEM((L, V, CHUNK_N), jnp.float32),
        pltpu.VMEM((L, V, CHUNK_N), jnp.float32),
        pltpu.SemaphoreType.DMA,
        pltpu.SemaphoreType.DMA,
        pltpu.SemaphoreType.DMA,
        pltpu.SemaphoreType.DMA,
        pltpu.SemaphoreType.DMA,
        pltpu.SemaphoreType.DMA,
    ],
    compiler_params=pltpu.CompilerParams(needs_layout_passes=False,
                                         use_tc_tiling_on_sc=True),
)
def _solve(x_hbm, out_hbm, buf0, buf1, buf2,
           si0, si1, si2, so0, so1, so2):
    bufs = (buf0, buf1, buf2)
    sin = (si0, si1, si2)
    sout = (so0, so1, so2)
    wid = lax.axis_index("s") * NC + lax.axis_index("c")
    base_n = wid * N_PER_W
    lanes = lax.iota(jnp.int32, 16)

    def src(ci):
        return x_hbm.at[:, :, pl.ds(base_n + ci * CHUNK_N, CHUNK_N)]

    def dst(ci):
        return out_hbm.at[:, :, pl.ds(base_n + ci * CHUNK_N, CHUNK_N)]

    def compute(buf):
        def group_body(gi, c2):
            g0 = gi * 16
            orall = _ci(0)
            for l in range(L):
                m = buf[l, 0, pl.ds(g0, 16)]
                b = _ci(1)
                for v in range(1, V):
                    xv = buf[l, v, pl.ds(g0, 16)]
                    gt = xv > m
                    b = jnp.where(gt, _ci(1 << v), b)
                    m = jnp.where(gt, xv, m)
                orall = orall | b
            nidx = lanes + g0
            one = _cf(1.0)
            zero = _cf(0.0)
            for l in range(1, L):
                miss = (orall & _ci(1 << (l - 1))) == _ci(0)
                for v in range(V):
                    val = one if v == (l - 1) else zero
                    plsc.store_scatter(buf, [_ci(l), _ci(v), nidx], val,
                                       mask=miss)
            return c2

        lax.fori_loop(0, GROUPS, group_body, 0)

    pltpu.async_copy(src(0), bufs[0], sin[0])
    pltpu.async_copy(src(1), bufs[1], sin[1])
    for ci in range(CHUNKS):
        b = ci % 3
        if ci + 2 < CHUNKS:
            nb = (ci + 2) % 3
            if ci >= 1:
                pltpu.make_async_copy(bufs[nb], dst(ci - 1), sout[nb]).wait()
            pltpu.async_copy(src(ci + 2), bufs[nb], sin[nb])
        pltpu.make_async_copy(src(ci), bufs[b], sin[b]).wait()
        compute(bufs[b])
        pltpu.async_copy(bufs[b], dst(ci), sout[b])
    for ci in range(max(0, CHUNKS - 3), CHUNKS):
        pltpu.make_async_copy(bufs[ci % 3], dst(ci), sout[ci % 3]).wait()


def kernel(memory_vb):
    xt = memory_vb.transpose(1, 2, 0)
    out = _solve(xt)
    return out.transpose(2, 0, 1)


# speedup vs baseline: 19.2870x; 1.0001x over previous
import functools

import jax
import jax.numpy as jnp
from jax import lax
from jax.experimental import pallas as pl
from jax.experimental.pallas import tpu as pltpu
from jax.experimental.pallas import tpu_sc as plsc

N = 65536
L = 10
V = 10
NC = 2
NS = 16
NW = NC * NS
N_PER_W = N // NW
CHUNK_N = 256
CHUNKS = N_PER_W // CHUNK_N
GROUPS = CHUNK_N // 16
NBUF = 3


def _ci(v):
    return lax.full((16,), v, jnp.int32)


def _cf(v):
    return lax.full((16,), v, jnp.float32)


@functools.partial(
    pl.kernel,
    out_type=jax.ShapeDtypeStruct((L, V, N), jnp.float32),
    mesh=plsc.VectorSubcoreMesh(core_axis_name="c", subcore_axis_name="s"),
    scratch_types=(
        [pltpu.VMEM((L, V, CHUNK_N), jnp.float32)] * NBUF
        + [pltpu.SemaphoreType.DMA] * (2 * NBUF)
    ),
    compiler_params=pltpu.CompilerParams(needs_layout_passes=False,
                                         use_tc_tiling_on_sc=True),
)
def _solve(x_hbm, out_hbm, *scratch):
    bufs = scratch[:NBUF]
    sin = scratch[NBUF:2 * NBUF]
    sout = scratch[2 * NBUF:]
    wid = lax.axis_index("s") * NC + lax.axis_index("c")
    base_n = wid * N_PER_W
    lanes = lax.iota(jnp.int32, 16)

    def src(ci):
        return x_hbm.at[:, :, pl.ds(base_n + ci * CHUNK_N, CHUNK_N)]

    def dst(ci):
        return out_hbm.at[:, :, pl.ds(base_n + ci * CHUNK_N, CHUNK_N)]

    def compute(buf):
        def group_body(gi, c2):
            g0 = gi * 16
            orall = _ci(0)
            for l in range(L):
                m = buf[l, 0, pl.ds(g0, 16)]
                b = _ci(1)
                for v in range(1, V):
                    xv = buf[l, v, pl.ds(g0, 16)]
                    gt = xv > m
                    b = jnp.where(gt, _ci(1 << v), b)
                    m = jnp.where(gt, xv, m)
                orall = orall | b
            nidx = lanes + g0
            one = _cf(1.0)
            zero = _cf(0.0)
            for l in range(1, L):
                miss = (orall & _ci(1 << (l - 1))) == _ci(0)
                for v in range(V):
                    val = one if v == (l - 1) else zero
                    plsc.store_scatter(buf, [_ci(l), _ci(v), nidx], val,
                                       mask=miss)
            return c2

        lax.fori_loop(0, GROUPS, group_body, 0)

    AHEAD = NBUF - 1
    for pi in range(AHEAD):
        pltpu.async_copy(src(pi), bufs[pi], sin[pi])
    for ci in range(CHUNKS):
        b = ci % NBUF
        if ci + AHEAD < CHUNKS:
            nb = (ci + AHEAD) % NBUF
            if ci >= 1:
                pltpu.make_async_copy(bufs[nb], dst(ci - 1), sout[nb]).wait()
            pltpu.async_copy(src(ci + AHEAD), bufs[nb], sin[nb])
        pltpu.make_async_copy(src(ci), bufs[b], sin[b]).wait()
        compute(bufs[b])
        pltpu.async_copy(bufs[b], dst(ci), sout[b])
    for ci in range(max(0, CHUNKS - AHEAD), CHUNKS):
        pltpu.make_async_copy(bufs[ci % NBUF], dst(ci), sout[ci % NBUF]).wait()


def kernel(memory_vb):
    xt = memory_vb.transpose(1, 2, 0)
    out = _solve(xt)
    return out.transpose(2, 0, 1)
